# baseline (speedup 1.0000x reference)
import functools

import numpy as np

import jax
import jax.numpy as jnp
from jax import lax
from jax.experimental import pallas as pl
from jax.experimental.pallas import tpu as pltpu
from jax.experimental.pallas import tpu_sc as plsc

B_ROWS = 4096
SEQ = 20
D = 768
N_LOOKUPS = B_ROWS * SEQ

NC = 2
NS = 16
NW = NC * NS
CHUNK = 64

N_CH = 4
CH = N_LOOKUPS // N_CH
ROWS_PER_W = CH // NW
NCHUNK = ROWS_PER_W // CHUNK

BM = 1024
BLOCKS_PER_CH = CH // BM

_PERM = np.arange(D).reshape(D // 32, 2, 16).transpose(0, 2, 1).reshape(D)


def _gather_body(x_hbm, table_hbm, out_hbm, idx_v, rows0, rows1, bf_v,
                 sem0, sem1):
    wid = lax.axis_index("s") * NC + lax.axis_index("c")
    base = wid * ROWS_PER_W
    bufs = (rows0, rows1)
    sems = (sem0, sem1)

    def gather_dma(k, p):
        return pltpu.make_async_copy(
            table_hbm.at[idx_v.at[pl.ds(k * CHUNK, CHUNK)]], bufs[p], sems[p])

    pltpu.sync_copy(x_hbm.at[pl.ds(base, ROWS_PER_W)], idx_v)
    gather_dma(0, 0).start()

    def pair(i, carry):
        for p in range(2):
            k = i * 2 + p

            @pl.when(k + 1 < NCHUNK)
            def _():
                gather_dma(k + 1, (p + 1) % 2).start()

            gather_dma(k, p).wait()
            src = bufs[p]

            def row(r, c2):
                for g in range(D // 32):
                    a = lax.bitcast_convert_type(src[r, pl.ds(32 * g, 16)], jnp.int32)
                    bb = lax.bitcast_convert_type(src[r, pl.ds(32 * g + 16, 16)], jnp.int32)
                    lo = lax.shift_right_logical(a + 0x8000, 16)
                    hi = jnp.bitwise_and(bb + 0x8000, jnp.int32(-0x10000))
                    bf_v[r, pl.ds(16 * g, 16)] = jnp.bitwise_or(lo, hi)
                return c2

            lax.fori_loop(0, CHUNK, row, 0)
            pltpu.sync_copy(bf_v, out_hbm.at[pl.ds(base + k * CHUNK, CHUNK)])
        return carry

    lax.fori_loop(0, NCHUNK // 2, pair, 0)


def _sc_gather(x_chunk, table):
    mesh = plsc.VectorSubcoreMesh(core_axis_name="c", subcore_axis_name="s")
    kern = functools.partial(
        pl.kernel,
        mesh=mesh,
        out_type=jax.ShapeDtypeStruct((CH, D // 2), jnp.int32),
        scratch_types=[
            pltpu.VMEM((ROWS_PER_W,), jnp.int32),
            pltpu.VMEM((CHUNK, D), jnp.float32),
            pltpu.VMEM((CHUNK, D), jnp.float32),
            pltpu.VMEM((CHUNK, D // 2), jnp.int32),
            pltpu.SemaphoreType.DMA,
            pltpu.SemaphoreType.DMA,
        ],
    )(_gather_body)
    return kern(x_chunk, table)


_SQRT_HALF = 0.7071067811865476


def _mm_first_body(emb_ref, wt_ref, b_ref, out_ref):
    h = jnp.dot(emb_ref[...], wt_ref[...], preferred_element_type=jnp.float32)
    h = h + b_ref[...]
    out_ref[...] = 0.5 * h * (1.0 + lax.erf(h * _SQRT_HALF))


def _mm_alias_body(buf_ref, emb_ref, wt_ref, b_ref, out_ref):
    del buf_ref
    h = jnp.dot(emb_ref[...], wt_ref[...], preferred_element_type=jnp.float32)
    h = h + b_ref[...]
    out_ref[...] = 0.5 * h * (1.0 + lax.erf(h * _SQRT_HALF))


def _tc_chunk(buf, emb_c, wt, bias, c):
    c_off = c * BLOCKS_PER_CH
    out_spec = pl.BlockSpec((BM, D), lambda i, _c=c_off: (i + _c, 0))
    emb_spec = pl.BlockSpec((BM, D), lambda i: (i, 0))
    wt_spec = pl.BlockSpec((D, D), lambda i: (0, 0))
    b_spec = pl.BlockSpec((1, D), lambda i: (0, 0))
    out_shape = jax.ShapeDtypeStruct((N_LOOKUPS, D), jnp.float32)
    if buf is None:
        return pl.pallas_call(
            _mm_first_body,
            grid=(BLOCKS_PER_CH,),
            in_specs=[emb_spec, wt_spec, b_spec],
            out_specs=out_spec,
            out_shape=out_shape,
        )(emb_c, wt, bias)
    return pl.pallas_call(
        _mm_alias_body,
        grid=(BLOCKS_PER_CH,),
        in_specs=[pl.BlockSpec(memory_space=pl.ANY),
                  emb_spec, wt_spec, b_spec],
        out_specs=out_spec,
        out_shape=out_shape,
        input_output_aliases={0: 0},
    )(buf, emb_c, wt, bias)


def kernel(x, table, W, b):
    x_t = x.astype(jnp.int32).T.reshape(-1)
    wt = W.T[_PERM].astype(jnp.bfloat16)
    bias = b.reshape(1, D)
    embs = [
        lax.bitcast_convert_type(
            _sc_gather(lax.slice(x_t, (c * CH,), ((c + 1) * CH,)), table),
            jnp.bfloat16).reshape(CH, D)
        for c in range(N_CH)]
    buf = None
    for c in range(N_CH):
        buf = _tc_chunk(buf, embs[c], wt, bias, c)
    return buf.reshape(SEQ, B_ROWS, D).transpose(1, 0, 2)


# speedup vs baseline: 1.1653x; 1.1653x over previous
import functools

import numpy as np

import jax
import jax.numpy as jnp
from jax import lax
from jax.experimental import pallas as pl
from jax.experimental.pallas import tpu as pltpu
from jax.experimental.pallas import tpu_sc as plsc

B_ROWS = 4096
SEQ = 20
D = 768
N_LOOKUPS = B_ROWS * SEQ

NC = 2
NS = 16
NW = NC * NS
CHUNK = 64

N_CH = 4
CH = N_LOOKUPS // N_CH
ROWS_PER_W = CH // NW
NCHUNK = ROWS_PER_W // CHUNK

BM = 1024
BLOCKS_PER_CH = CH // BM

_PERM = np.arange(D).reshape(D // 32, 2, 16).transpose(0, 2, 1).reshape(D)


def _gather_body(x_hbm, table_hbm, out_hbm, idx_v, rows0, rows1, bf_v,
                 sem0, sem1):
    wid = lax.axis_index("s") * NC + lax.axis_index("c")
    base = wid * ROWS_PER_W
    bufs = (rows0, rows1)
    sems = (sem0, sem1)

    def gather_dma(k, p):
        return pltpu.make_async_copy(
            table_hbm.at[idx_v.at[pl.ds(k * CHUNK, CHUNK)]], bufs[p], sems[p])

    pltpu.sync_copy(x_hbm.at[pl.ds(base, ROWS_PER_W)], idx_v)
    gather_dma(0, 0).start()

    def pair(i, carry):
        for p in range(2):
            k = i * 2 + p

            @pl.when(k + 1 < NCHUNK)
            def _():
                gather_dma(k + 1, (p + 1) % 2).start()

            gather_dma(k, p).wait()
            src = bufs[p]

            @plsc.parallel_loop(0, CHUNK, unroll=4)
            def _row(r):
                for g in range(D // 32):
                    a = lax.bitcast_convert_type(src[r, pl.ds(32 * g, 16)], jnp.int32)
                    bb = lax.bitcast_convert_type(src[r, pl.ds(32 * g + 16, 16)], jnp.int32)
                    lo = lax.shift_right_logical(a + 0x8000, 16)
                    hi = jnp.bitwise_and(bb + 0x8000, jnp.int32(-0x10000))
                    bf_v[r, pl.ds(16 * g, 16)] = jnp.bitwise_or(lo, hi)
            pltpu.sync_copy(bf_v, out_hbm.at[pl.ds(base + k * CHUNK, CHUNK)])
        return carry

    lax.fori_loop(0, NCHUNK // 2, pair, 0)


def _sc_gather(x_chunk, table):
    mesh = plsc.VectorSubcoreMesh(core_axis_name="c", subcore_axis_name="s")
    kern = functools.partial(
        pl.kernel,
        mesh=mesh,
        out_type=jax.ShapeDtypeStruct((CH, D // 2), jnp.int32),
        scratch_types=[
            pltpu.VMEM((ROWS_PER_W,), jnp.int32),
            pltpu.VMEM((CHUNK, D), jnp.float32),
            pltpu.VMEM((CHUNK, D), jnp.float32),
            pltpu.VMEM((CHUNK, D // 2), jnp.int32),
            pltpu.SemaphoreType.DMA,
            pltpu.SemaphoreType.DMA,
        ],
    )(_gather_body)
    return kern(x_chunk, table)


_SQRT_HALF = 0.7071067811865476


def _mm_first_body(emb_ref, wt_ref, b_ref, out_ref):
    h = jnp.dot(emb_ref[...], wt_ref[...], preferred_element_type=jnp.float32)
    h = h + b_ref[...]
    out_ref[...] = 0.5 * h * (1.0 + lax.erf(h * _SQRT_HALF))


def _mm_alias_body(buf_ref, emb_ref, wt_ref, b_ref, out_ref):
    del buf_ref
    h = jnp.dot(emb_ref[...], wt_ref[...], preferred_element_type=jnp.float32)
    h = h + b_ref[...]
    out_ref[...] = 0.5 * h * (1.0 + lax.erf(h * _SQRT_HALF))


def _tc_chunk(buf, emb_c, wt, bias, c):
    c_off = c * BLOCKS_PER_CH
    out_spec = pl.BlockSpec((BM, D), lambda i, _c=c_off: (i + _c, 0))
    emb_spec = pl.BlockSpec((BM, D), lambda i: (i, 0))
    wt_spec = pl.BlockSpec((D, D), lambda i: (0, 0))
    b_spec = pl.BlockSpec((1, D), lambda i: (0, 0))
    out_shape = jax.ShapeDtypeStruct((N_LOOKUPS, D), jnp.float32)
    if buf is None:
        return pl.pallas_call(
            _mm_first_body,
            grid=(BLOCKS_PER_CH,),
            in_specs=[emb_spec, wt_spec, b_spec],
            out_specs=out_spec,
            out_shape=out_shape,
        )(emb_c, wt, bias)
    return pl.pallas_call(
        _mm_alias_body,
        grid=(BLOCKS_PER_CH,),
        in_specs=[pl.BlockSpec(memory_space=pl.ANY),
                  emb_spec, wt_spec, b_spec],
        out_specs=out_spec,
        out_shape=out_shape,
        input_output_aliases={0: 0},
    )(buf, emb_c, wt, bias)


def kernel(x, table, W, b):
    x_t = x.astype(jnp.int32).T.reshape(-1)
    wt = W.T[_PERM].astype(jnp.bfloat16)
    bias = b.reshape(1, D)
    embs = [
        lax.bitcast_convert_type(
            _sc_gather(lax.slice(x_t, (c * CH,), ((c + 1) * CH,)), table),
            jnp.bfloat16).reshape(CH, D)
        for c in range(N_CH)]
    buf = None
    for c in range(N_CH):
        buf = _tc_chunk(buf, embs[c], wt, bias, c)
    return buf.reshape(SEQ, B_ROWS, D).transpose(1, 0, 2)


# speedup vs baseline: 4.7929x; 4.1129x over previous
import functools

import jax
import jax.numpy as jnp
from jax import lax
from jax.experimental import pallas as pl
from jax.experimental.pallas import tpu as pltpu
from jax.experimental.pallas import tpu_sc as plsc

B_ROWS = 4096
SEQ = 20
D = 768
N_LOOKUPS = B_ROWS * SEQ

NC = 2
NS = 16
NW = NC * NS
CHUNK = 64

CH_SIZES = (8192, 16384, 16384, 16384, 16384, 8192)
assert sum(CH_SIZES) == N_LOOKUPS

BM = 1024


def _make_gather_body(rows_per_w, nchunk):
    def _gather_body(x_hbm, table_hbm, out_hbm, idx_v, rows0, rows1,
                     sem0, sem1):
        wid = lax.axis_index("s") * NC + lax.axis_index("c")
        base = wid * rows_per_w
        bufs = (rows0, rows1)
        sems = (sem0, sem1)

        def gather_dma(k, p):
            return pltpu.make_async_copy(
                table_hbm.at[idx_v.at[pl.ds(k * CHUNK, CHUNK)]],
                bufs[p], sems[p])

        pltpu.sync_copy(x_hbm.at[pl.ds(base, rows_per_w)], idx_v)
        gather_dma(0, 0).start()

        def pair(i, carry):
            for p in range(2):
                k = i * 2 + p

                @pl.when(k + 1 < nchunk)
                def _():
                    gather_dma(k + 1, (p + 1) % 2).start()

                gather_dma(k, p).wait()
                pltpu.sync_copy(
                    bufs[p], out_hbm.at[pl.ds(base + k * CHUNK, CHUNK)])
            return carry

        lax.fori_loop(0, nchunk // 2, pair, 0)

    return _gather_body


def _sc_gather(x_chunk, table, ch):
    rows_per_w = ch // NW
    nchunk = rows_per_w // CHUNK
    mesh = plsc.VectorSubcoreMesh(core_axis_name="c", subcore_axis_name="s")
    kern = functools.partial(
        pl.kernel,
        mesh=mesh,
        out_type=jax.ShapeDtypeStruct((ch, D), jnp.float32),
        scratch_types=[
            pltpu.VMEM((rows_per_w,), jnp.int32),
            pltpu.VMEM((CHUNK, D), jnp.float32),
            pltpu.VMEM((CHUNK, D), jnp.float32),
            pltpu.SemaphoreType.DMA,
            pltpu.SemaphoreType.DMA,
        ],
    )(_make_gather_body(rows_per_w, nchunk))
    return kern(x_chunk, table)


_SQRT_HALF = 0.7071067811865476


def _mm_first_body(emb_ref, wt_ref, b_ref, out_ref):
    h = jnp.dot(emb_ref[...], wt_ref[...], preferred_element_type=jnp.float32)
    h = h + b_ref[...]
    out_ref[...] = 0.5 * h * (1.0 + lax.erf(h * _SQRT_HALF))


def _mm_alias_body(buf_ref, emb_ref, wt_ref, b_ref, out_ref):
    del buf_ref
    h = jnp.dot(emb_ref[...], wt_ref[...], preferred_element_type=jnp.float32)
    h = h + b_ref[...]
    out_ref[...] = 0.5 * h * (1.0 + lax.erf(h * _SQRT_HALF))


def _tc_chunk(buf, emb_c, wt, bias, row_off):
    blocks = emb_c.shape[0] // BM
    block_off = row_off // BM
    out_spec = pl.BlockSpec((BM, D), lambda i, _c=block_off: (i + _c, 0))
    emb_spec = pl.BlockSpec((BM, D), lambda i: (i, 0))
    wt_spec = pl.BlockSpec((D, D), lambda i: (0, 0))
    b_spec = pl.BlockSpec((1, D), lambda i: (0, 0))
    out_shape = jax.ShapeDtypeStruct((N_LOOKUPS, D), jnp.float32)
    if buf is None:
        return pl.pallas_call(
            _mm_first_body,
            grid=(blocks,),
            in_specs=[emb_spec, wt_spec, b_spec],
            out_specs=out_spec,
            out_shape=out_shape,
        )(emb_c, wt, bias)
    return pl.pallas_call(
        _mm_alias_body,
        grid=(blocks,),
        in_specs=[pl.BlockSpec(memory_space=pl.ANY),
                  emb_spec, wt_spec, b_spec],
        out_specs=out_spec,
        out_shape=out_shape,
        input_output_aliases={0: 0},
    )(buf, emb_c, wt, bias)


def kernel(x, table, W, b):
    x_t = x.astype(jnp.int32).T.reshape(-1)
    wt = W.T
    bias = b.reshape(1, D)
    offs = [0]
    for ch in CH_SIZES:
        offs.append(offs[-1] + ch)
    embs = [_sc_gather(lax.slice(x_t, (offs[c],), (offs[c + 1],)), table,
                       CH_SIZES[c])
            for c in range(len(CH_SIZES))]
    buf = None
    for c in range(len(CH_SIZES)):
        buf = _tc_chunk(buf, embs[c], wt, bias, offs[c])
    return buf.reshape(SEQ, B_ROWS, D).transpose(1, 0, 2)


# speedup vs baseline: 4.8122x; 1.0040x over previous
import functools

import jax
import jax.numpy as jnp
from jax import lax
from jax.experimental import pallas as pl
from jax.experimental.pallas import tpu as pltpu
from jax.experimental.pallas import tpu_sc as plsc

B_ROWS = 4096
SEQ = 20
D = 768
N_LOOKUPS = B_ROWS * SEQ

NC = 2
NS = 16
NW = NC * NS
CHUNK = 64

CH_SIZES = (8192, 28672, 28672, 16384)
assert sum(CH_SIZES) == N_LOOKUPS

BM = 1024


def _make_gather_body(rows_per_w, nchunk):
    def _gather_body(x_hbm, table_hbm, out_hbm, idx_v, rows0, rows1,
                     sem0, sem1):
        wid = lax.axis_index("s") * NC + lax.axis_index("c")
        base = wid * rows_per_w
        bufs = (rows0, rows1)
        sems = (sem0, sem1)

        def gather_dma(k, p):
            return pltpu.make_async_copy(
                table_hbm.at[idx_v.at[pl.ds(k * CHUNK, CHUNK)]],
                bufs[p], sems[p])

        pltpu.sync_copy(x_hbm.at[pl.ds(base, rows_per_w)], idx_v)
        gather_dma(0, 0).start()

        def pair(i, carry):
            for p in range(2):
                k = i * 2 + p

                @pl.when(k + 1 < nchunk)
                def _():
                    gather_dma(k + 1, (p + 1) % 2).start()

                gather_dma(k, p).wait()
                pltpu.sync_copy(
                    bufs[p], out_hbm.at[pl.ds(base + k * CHUNK, CHUNK)])
            return carry

        lax.fori_loop(0, nchunk // 2, pair, 0)

    return _gather_body


def _sc_gather(x_chunk, table, ch):
    rows_per_w = ch // NW
    nchunk = rows_per_w // CHUNK
    mesh = plsc.VectorSubcoreMesh(core_axis_name="c", subcore_axis_name="s")
    kern = functools.partial(
        pl.kernel,
        mesh=mesh,
        out_type=jax.ShapeDtypeStruct((ch, D), jnp.float32),
        scratch_types=[
            pltpu.VMEM((rows_per_w,), jnp.int32),
            pltpu.VMEM((CHUNK, D), jnp.float32),
            pltpu.VMEM((CHUNK, D), jnp.float32),
            pltpu.SemaphoreType.DMA,
            pltpu.SemaphoreType.DMA,
        ],
    )(_make_gather_body(rows_per_w, nchunk))
    return kern(x_chunk, table)


_SQRT_HALF = 0.7071067811865476


def _mm_first_body(emb_ref, wt_ref, b_ref, out_ref):
    h = jnp.dot(emb_ref[...], wt_ref[...], preferred_element_type=jnp.float32)
    h = h + b_ref[...]
    out_ref[...] = 0.5 * h * (1.0 + lax.erf(h * _SQRT_HALF))


def _mm_alias_body(buf_ref, emb_ref, wt_ref, b_ref, out_ref):
    del buf_ref
    h = jnp.dot(emb_ref[...], wt_ref[...], preferred_element_type=jnp.float32)
    h = h + b_ref[...]
    out_ref[...] = 0.5 * h * (1.0 + lax.erf(h * _SQRT_HALF))


def _tc_chunk(buf, emb_c, wt, bias, row_off):
    blocks = emb_c.shape[0] // BM
    block_off = row_off // BM
    out_spec = pl.BlockSpec((BM, D), lambda i, _c=block_off: (i + _c, 0))
    emb_spec = pl.BlockSpec((BM, D), lambda i: (i, 0))
    wt_spec = pl.BlockSpec((D, D), lambda i: (0, 0))
    b_spec = pl.BlockSpec((1, D), lambda i: (0, 0))
    out_shape = jax.ShapeDtypeStruct((N_LOOKUPS, D), jnp.float32)
    if buf is None:
        return pl.pallas_call(
            _mm_first_body,
            grid=(blocks,),
            in_specs=[emb_spec, wt_spec, b_spec],
            out_specs=out_spec,
            out_shape=out_shape,
        )(emb_c, wt, bias)
    return pl.pallas_call(
        _mm_alias_body,
        grid=(blocks,),
        in_specs=[pl.BlockSpec(memory_space=pl.ANY),
                  emb_spec, wt_spec, b_spec],
        out_specs=out_spec,
        out_shape=out_shape,
        input_output_aliases={0: 0},
    )(buf, emb_c, wt, bias)


def kernel(x, table, W, b):
    x_t = x.astype(jnp.int32).T.reshape(-1)
    wt = W.T
    bias = b.reshape(1, D)
    offs = [0]
    for ch in CH_SIZES:
        offs.append(offs[-1] + ch)
    embs = [_sc_gather(lax.slice(x_t, (offs[c],), (offs[c + 1],)), table,
                       CH_SIZES[c])
            for c in range(len(CH_SIZES))]
    buf = None
    for c in range(len(CH_SIZES)):
        buf = _tc_chunk(buf, embs[c], wt, bias, offs[c])
    return buf.reshape(SEQ, B_ROWS, D).transpose(1, 0, 2)


# speedup vs baseline: 4.8890x; 1.0160x over previous
import functools

import jax
import jax.numpy as jnp
from jax import lax
from jax.experimental import pallas as pl
from jax.experimental.pallas import tpu as pltpu
from jax.experimental.pallas import tpu_sc as plsc

B_ROWS = 4096
SEQ = 20
D = 768
N_LOOKUPS = B_ROWS * SEQ

NC = 2
NS = 16
NW = NC * NS
CHUNK = 80

CH_SIZES = (20480, 20480, 20480, 20480)
assert sum(CH_SIZES) == N_LOOKUPS

BM = 1024


def _make_gather_body(rows_per_w, nchunk):
    def _gather_body(x_hbm, table_hbm, out_hbm, idx_v, rows0, rows1,
                     sem0, sem1):
        wid = lax.axis_index("s") * NC + lax.axis_index("c")
        base = wid * rows_per_w
        bufs = (rows0, rows1)
        sems = (sem0, sem1)

        def gather_dma(k, p):
            return pltpu.make_async_copy(
                table_hbm.at[idx_v.at[pl.ds(k * CHUNK, CHUNK)]],
                bufs[p], sems[p])

        pltpu.sync_copy(x_hbm.at[pl.ds(base, rows_per_w)], idx_v)
        gather_dma(0, 0).start()

        def pair(i, carry):
            for p in range(2):
                k = i * 2 + p

                @pl.when(k + 1 < nchunk)
                def _():
                    gather_dma(k + 1, (p + 1) % 2).start()

                gather_dma(k, p).wait()
                pltpu.sync_copy(
                    bufs[p], out_hbm.at[pl.ds(base + k * CHUNK, CHUNK)])
            return carry

        lax.fori_loop(0, nchunk // 2, pair, 0)

    return _gather_body


def _sc_gather(x_chunk, table, ch):
    rows_per_w = ch // NW
    nchunk = rows_per_w // CHUNK
    mesh = plsc.VectorSubcoreMesh(core_axis_name="c", subcore_axis_name="s")
    kern = functools.partial(
        pl.kernel,
        mesh=mesh,
        out_type=jax.ShapeDtypeStruct((ch, D), jnp.float32),
        scratch_types=[
            pltpu.VMEM((rows_per_w,), jnp.int32),
            pltpu.VMEM((CHUNK, D), jnp.float32),
            pltpu.VMEM((CHUNK, D), jnp.float32),
            pltpu.SemaphoreType.DMA,
            pltpu.SemaphoreType.DMA,
        ],
    )(_make_gather_body(rows_per_w, nchunk))
    return kern(x_chunk, table)


_SQRT_HALF = 0.7071067811865476


def _mm_first_body(emb_ref, wt_ref, b_ref, out_ref):
    h = jnp.dot(emb_ref[...], wt_ref[...], preferred_element_type=jnp.float32)
    h = h + b_ref[...]
    out_ref[...] = 0.5 * h * (1.0 + lax.erf(h * _SQRT_HALF))


def _mm_alias_body(buf_ref, emb_ref, wt_ref, b_ref, out_ref):
    del buf_ref
    h = jnp.dot(emb_ref[...], wt_ref[...], preferred_element_type=jnp.float32)
    h = h + b_ref[...]
    out_ref[...] = 0.5 * h * (1.0 + lax.erf(h * _SQRT_HALF))


def _tc_chunk(buf, emb_c, wt, bias, row_off):
    blocks = emb_c.shape[0] // BM
    block_off = row_off // BM
    out_spec = pl.BlockSpec((BM, D), lambda i, _c=block_off: (i + _c, 0))
    emb_spec = pl.BlockSpec((BM, D), lambda i: (i, 0))
    wt_spec = pl.BlockSpec((D, D), lambda i: (0, 0))
    b_spec = pl.BlockSpec((1, D), lambda i: (0, 0))
    out_shape = jax.ShapeDtypeStruct((N_LOOKUPS, D), jnp.float32)
    if buf is None:
        return pl.pallas_call(
            _mm_first_body,
            grid=(blocks,),
            in_specs=[emb_spec, wt_spec, b_spec],
            out_specs=out_spec,
            out_shape=out_shape,
        )(emb_c, wt, bias)
    return pl.pallas_call(
        _mm_alias_body,
        grid=(blocks,),
        in_specs=[pl.BlockSpec(memory_space=pl.ANY),
                  emb_spec, wt_spec, b_spec],
        out_specs=out_spec,
        out_shape=out_shape,
        input_output_aliases={0: 0},
    )(buf, emb_c, wt, bias)


def kernel(x, table, W, b):
    x_t = x.astype(jnp.int32).T.reshape(-1)
    wt = W.T
    bias = b.reshape(1, D)
    offs = [0]
    for ch in CH_SIZES:
        offs.append(offs[-1] + ch)
    embs = [_sc_gather(lax.slice(x_t, (offs[c],), (offs[c + 1],)), table,
                       CH_SIZES[c])
            for c in range(len(CH_SIZES))]
    buf = None
    for c in range(len(CH_SIZES)):
        buf = _tc_chunk(buf, embs[c], wt, bias, offs[c])
    return buf.reshape(SEQ, B_ROWS, D).transpose(1, 0, 2)


# speedup vs baseline: 4.9364x; 1.0097x over previous
import functools

import jax
import jax.numpy as jnp
from jax import lax
from jax.experimental import pallas as pl
from jax.experimental.pallas import tpu as pltpu
from jax.experimental.pallas import tpu_sc as plsc

B_ROWS = 4096
SEQ = 20
D = 768
N_LOOKUPS = B_ROWS * SEQ

NC = 2
NS = 16
NW = NC * NS
CHUNK = 80

CH_SIZES = (20480, 20480, 20480, 20480)
assert sum(CH_SIZES) == N_LOOKUPS

BM = 2048


def _make_gather_body(rows_per_w, nchunk):
    def _gather_body(x_hbm, table_hbm, out_hbm, idx_v, rows0, rows1,
                     sem0, sem1):
        wid = lax.axis_index("s") * NC + lax.axis_index("c")
        base = wid * rows_per_w
        bufs = (rows0, rows1)
        sems = (sem0, sem1)

        def gather_dma(k, p):
            return pltpu.make_async_copy(
                table_hbm.at[idx_v.at[pl.ds(k * CHUNK, CHUNK)]],
                bufs[p], sems[p])

        pltpu.sync_copy(x_hbm.at[pl.ds(base, rows_per_w)], idx_v)
        gather_dma(0, 0).start()

        def pair(i, carry):
            for p in range(2):
                k = i * 2 + p

                @pl.when(k + 1 < nchunk)
                def _():
                    gather_dma(k + 1, (p + 1) % 2).start()

                gather_dma(k, p).wait()
                pltpu.sync_copy(
                    bufs[p], out_hbm.at[pl.ds(base + k * CHUNK, CHUNK)])
            return carry

        lax.fori_loop(0, nchunk // 2, pair, 0)

    return _gather_body


def _sc_gather(x_chunk, table, ch):
    rows_per_w = ch // NW
    nchunk = rows_per_w // CHUNK
    mesh = plsc.VectorSubcoreMesh(core_axis_name="c", subcore_axis_name="s")
    kern = functools.partial(
        pl.kernel,
        mesh=mesh,
        out_type=jax.ShapeDtypeStruct((ch, D), jnp.float32),
        scratch_types=[
            pltpu.VMEM((rows_per_w,), jnp.int32),
            pltpu.VMEM((CHUNK, D), jnp.float32),
            pltpu.VMEM((CHUNK, D), jnp.float32),
            pltpu.SemaphoreType.DMA,
            pltpu.SemaphoreType.DMA,
        ],
    )(_make_gather_body(rows_per_w, nchunk))
    return kern(x_chunk, table)


_SQRT_HALF = 0.7071067811865476


def _mm_first_body(emb_ref, wt_ref, b_ref, out_ref):
    h = jnp.dot(emb_ref[...], wt_ref[...], preferred_element_type=jnp.float32)
    h = h + b_ref[...]
    out_ref[...] = 0.5 * h * (1.0 + lax.erf(h * _SQRT_HALF))


def _mm_alias_body(buf_ref, emb_ref, wt_ref, b_ref, out_ref):
    del buf_ref
    h = jnp.dot(emb_ref[...], wt_ref[...], preferred_element_type=jnp.float32)
    h = h + b_ref[...]
    out_ref[...] = 0.5 * h * (1.0 + lax.erf(h * _SQRT_HALF))


def _tc_chunk(buf, emb_c, wt, bias, row_off):
    blocks = emb_c.shape[0] // BM
    block_off = row_off // BM
    out_spec = pl.BlockSpec((BM, D), lambda i, _c=block_off: (i + _c, 0))
    emb_spec = pl.BlockSpec((BM, D), lambda i: (i, 0))
    wt_spec = pl.BlockSpec((D, D), lambda i: (0, 0))
    b_spec = pl.BlockSpec((1, D), lambda i: (0, 0))
    out_shape = jax.ShapeDtypeStruct((N_LOOKUPS, D), jnp.float32)
    if buf is None:
        return pl.pallas_call(
            _mm_first_body,
            grid=(blocks,),
            in_specs=[emb_spec, wt_spec, b_spec],
            out_specs=out_spec,
            out_shape=out_shape,
        )(emb_c, wt, bias)
    return pl.pallas_call(
        _mm_alias_body,
        grid=(blocks,),
        in_specs=[pl.BlockSpec(memory_space=pl.ANY),
                  emb_spec, wt_spec, b_spec],
        out_specs=out_spec,
        out_shape=out_shape,
        input_output_aliases={0: 0},
    )(buf, emb_c, wt, bias)


def kernel(x, table, W, b):
    x_t = x.astype(jnp.int32).T.reshape(-1)
    wt = W.T
    bias = b.reshape(1, D)
    offs = [0]
    for ch in CH_SIZES:
        offs.append(offs[-1] + ch)
    embs = [_sc_gather(lax.slice(x_t, (offs[c],), (offs[c + 1],)), table,
                       CH_SIZES[c])
            for c in range(len(CH_SIZES))]
    buf = None
    for c in range(len(CH_SIZES)):
        buf = _tc_chunk(buf, embs[c], wt, bias, offs[c])
    return buf.reshape(SEQ, B_ROWS, D).transpose(1, 0, 2)


# speedup vs baseline: 5.0453x; 1.0221x over previous
import functools

import jax
import jax.numpy as jnp
from jax import lax
from jax.experimental import pallas as pl
from jax.experimental.pallas import tpu as pltpu
from jax.experimental.pallas import tpu_sc as plsc

B_ROWS = 4096
SEQ = 20
D = 768
N_LOOKUPS = B_ROWS * SEQ

NC = 2
NS = 16
NW = NC * NS
CHUNK = 80

CH_SIZES = (20480, 20480, 20480, 20480)
assert sum(CH_SIZES) == N_LOOKUPS

BM = 4096


def _make_gather_body(rows_per_w, nchunk):
    def _gather_body(x_hbm, table_hbm, out_hbm, idx_v, rows0, rows1,
                     sem0, sem1):
        wid = lax.axis_index("s") * NC + lax.axis_index("c")
        base = wid * rows_per_w
        bufs = (rows0, rows1)
        sems = (sem0, sem1)

        def gather_dma(k, p):
            return pltpu.make_async_copy(
                table_hbm.at[idx_v.at[pl.ds(k * CHUNK, CHUNK)]],
                bufs[p], sems[p])

        pltpu.sync_copy(x_hbm.at[pl.ds(base, rows_per_w)], idx_v)
        gather_dma(0, 0).start()

        def pair(i, carry):
            for p in range(2):
                k = i * 2 + p

                @pl.when(k + 1 < nchunk)
                def _():
                    gather_dma(k + 1, (p + 1) % 2).start()

                gather_dma(k, p).wait()
                pltpu.sync_copy(
                    bufs[p], out_hbm.at[pl.ds(base + k * CHUNK, CHUNK)])
            return carry

        lax.fori_loop(0, nchunk // 2, pair, 0)

    return _gather_body


def _sc_gather(x_chunk, table, ch):
    rows_per_w = ch // NW
    nchunk = rows_per_w // CHUNK
    mesh = plsc.VectorSubcoreMesh(core_axis_name="c", subcore_axis_name="s")
    kern = functools.partial(
        pl.kernel,
        mesh=mesh,
        out_type=jax.ShapeDtypeStruct((ch, D), jnp.float32),
        scratch_types=[
            pltpu.VMEM((rows_per_w,), jnp.int32),
            pltpu.VMEM((CHUNK, D), jnp.float32),
            pltpu.VMEM((CHUNK, D), jnp.float32),
            pltpu.SemaphoreType.DMA,
            pltpu.SemaphoreType.DMA,
        ],
    )(_make_gather_body(rows_per_w, nchunk))
    return kern(x_chunk, table)


_SQRT_HALF = 0.7071067811865476


def _mm_first_body(emb_ref, wt_ref, b_ref, out_ref):
    h = jnp.dot(emb_ref[...], wt_ref[...], preferred_element_type=jnp.float32)
    h = h + b_ref[...]
    out_ref[...] = 0.5 * h * (1.0 + lax.erf(h * _SQRT_HALF))


def _mm_alias_body(buf_ref, emb_ref, wt_ref, b_ref, out_ref):
    del buf_ref
    h = jnp.dot(emb_ref[...], wt_ref[...], preferred_element_type=jnp.float32)
    h = h + b_ref[...]
    out_ref[...] = 0.5 * h * (1.0 + lax.erf(h * _SQRT_HALF))


def _tc_chunk(buf, emb_c, wt, bias, row_off):
    blocks = emb_c.shape[0] // BM
    block_off = row_off // BM
    out_spec = pl.BlockSpec((BM, D), lambda i, _c=block_off: (i + _c, 0))
    emb_spec = pl.BlockSpec((BM, D), lambda i: (i, 0))
    wt_spec = pl.BlockSpec((D, D), lambda i: (0, 0))
    b_spec = pl.BlockSpec((1, D), lambda i: (0, 0))
    out_shape = jax.ShapeDtypeStruct((N_LOOKUPS, D), jnp.float32)
    if buf is None:
        return pl.pallas_call(
            _mm_first_body,
            grid=(blocks,),
            in_specs=[emb_spec, wt_spec, b_spec],
            out_specs=out_spec,
            out_shape=out_shape,
        )(emb_c, wt, bias)
    return pl.pallas_call(
        _mm_alias_body,
        grid=(blocks,),
        in_specs=[pl.BlockSpec(memory_space=pl.ANY),
                  emb_spec, wt_spec, b_spec],
        out_specs=out_spec,
        out_shape=out_shape,
        input_output_aliases={0: 0},
    )(buf, emb_c, wt, bias)


def kernel(x, table, W, b):
    x_t = x.astype(jnp.int32).T.reshape(-1)
    wt = W.T
    bias = b.reshape(1, D)
    offs = [0]
    for ch in CH_SIZES:
        offs.append(offs[-1] + ch)
    embs = [_sc_gather(lax.slice(x_t, (offs[c],), (offs[c + 1],)), table,
                       CH_SIZES[c])
            for c in range(len(CH_SIZES))]
    buf = None
    for c in range(len(CH_SIZES)):
        buf = _tc_chunk(buf, embs[c], wt, bias, offs[c])
    return buf.reshape(SEQ, B_ROWS, D).transpose(1, 0, 2)
